# baseline (speedup 1.0000x reference)
import functools

import jax
import jax.numpy as jnp
from jax import lax
from jax.experimental import pallas as pl
from jax.experimental.pallas import tpu as pltpu
from jax.experimental.pallas import tpu_sc as plsc

B, N = 1, 6
C, H, W = 64, 32, 88
GH, GW = 128, 128
K = GH * GW
NGP = 9
IMG_H, IMG_W = 256.0, 704.0
EPS = 1e-05
BEV_Z = -1.0

NC, NS = 2, 16
NW = NC * NS
P_CHUNK = 512
SUB = 128
NSUB = P_CHUNK // SUB
N_UNITS = N * NGP


def _tc_index_body(i_ref, e_ref, p_ref, inds_ref, mask_ref):
    n = pl.program_id(0)
    p2i = jnp.matmul(i_ref[0], e_ref[0])
    sp = lax.dot_general(p2i, p_ref[...], (((1,), (1,)), ((), ())))
    sx, sy, sz = sp[0:1], sp[1:2], sp[2:3]

    zed = jnp.maximum(sz, EPS)
    xn = sx / zed / IMG_W
    yn = sy / zed / IMG_H
    mask_ref[0] = ((sz > EPS)
                   & (xn > 0.0) & (xn < 1.0)
                   & (yn > 0.0) & (yn < 1.0))

    u = jnp.round(xn * float(W))
    v = jnp.round(yn * float(H))
    cam_off = (n * (H * W)).astype(jnp.float32)
    for g in range(NGP):
        dx = float(g % 3 - 1)
        dy = float(g // 3 - 1)
        xi = jnp.clip(u + dx, 0.0, float(W - 1))
        yi = jnp.clip(v + dy, 0.0, float(H - 1))
        ind = xi + yi * float(W) + cam_off
        inds_ref[0, g:g + 1, :] = ind.astype(jnp.int32)


def _tc_indices(intrin, e_mats, pts4):
    return pl.pallas_call(
        _tc_index_body,
        grid=(N,),
        in_specs=[
            pl.BlockSpec((1, 4, 4), lambda n: (n, 0, 0)),
            pl.BlockSpec((1, 4, 4), lambda n: (n, 0, 0)),
            pl.BlockSpec((K, 4), lambda n: (0, 0)),
        ],
        out_specs=[
            pl.BlockSpec((1, NGP, K), lambda n: (n, 0, 0)),
            pl.BlockSpec((1, 1, K), lambda n: (n, 0, 0)),
        ],
        out_shape=[
            jax.ShapeDtypeStruct((N, NGP, K), jnp.int32),
            jax.ShapeDtypeStruct((N, 1, K), jnp.bool_),
        ],
    )(intrin, e_mats, pts4)


def _sc_gather_body(tab_hbm, inds_hbm, out_hbm, idx_v, buf_v, sem):
    wid = lax.axis_index("s") * NC + lax.axis_index("c")
    r0 = wid * NSUB
    p0 = wid * P_CHUNK

    @pl.loop(0, N_UNITS)
    def _unit(t):
        cam = t // NGP
        g = lax.rem(t, NGP)
        pltpu.sync_copy(inds_hbm.at[cam, g, pl.ds(r0, NSUB)], idx_v)
        copies = []
        for j in range(NSUB):
            copies.append(pltpu.async_copy(
                tab_hbm.at[idx_v.at[j]],
                buf_v.at[pl.ds(j * SUB, SUB)],
                sem,
            ))
        for cp in copies:
            cp.wait()
        pltpu.sync_copy(buf_v, out_hbm.at[cam, g, pl.ds(p0, P_CHUNK)])


def _sc_gather(imtab, inds):
    call = pl.kernel(
        _sc_gather_body,
        out_type=jax.ShapeDtypeStruct((N, NGP, K, C), jnp.float32),
        mesh=plsc.VectorSubcoreMesh(core_axis_name="c", subcore_axis_name="s",
                                    num_cores=NC, num_subcores=NS),
        scratch_types=[
            pltpu.VMEM((NSUB, SUB), jnp.int32),
            pltpu.VMEM((P_CHUNK, C), jnp.float32),
            pltpu.SemaphoreType.DMA,
        ],
        compiler_params=pltpu.CompilerParams(use_tc_tiling_on_sc=False),
    )
    return call(imtab, inds)


def kernel(bev_grids, images, I, E):
    imtab = jnp.transpose(images, (0, 2, 3, 1)).reshape(N * H * W, C)
    intrin = jnp.pad(I, ((0, 0), (0, 0), (0, 1), (0, 1)))
    intrin = intrin.at[..., 3, 3].set(1.0)
    xc = bev_grids[0].reshape(K, 1)
    yc = bev_grids[1].reshape(K, 1)
    pts4 = jnp.concatenate(
        [xc, yc, jnp.full_like(xc, BEV_Z), jnp.ones_like(xc)], axis=1)
    inds, mask = _tc_indices(intrin[0], E.reshape(N, 4, 4), pts4)
    feats = _sc_gather(imtab, inds.reshape(N, NGP, GH, GW))
    feats = feats.transpose(0, 2, 1, 3)
    return (feats.reshape(B, N, K, NGP, C),
            mask.reshape(B, N, K, 1))
